# baseline (speedup 1.0000x reference)
import numpy as np
import jax
import jax.numpy as jnp
from jax import lax
from jax.experimental import pallas as pl
from jax.experimental.pallas import tpu as pltpu
from jax.experimental.pallas import tpu_sc as plsc

_VOCAB, _DMODEL, _BATCH, _SEQ = 100000, 1024, 4, 2048
_NC, _NS, _L = 2, 16, 16
_NW = _NC * _NS
_B = _BATCH * _SEQ
_SPW = _SEQ // _NW
_CH = 32
_NSTEP = (_SPW // _CH) * _BATCH
_NGRP = _DMODEL // (2 * _L)
_NU = _SEQ // _CH
_BWORDS = _CH * _DMODEL // 2


def _pe_factors():
    import ml_dtypes
    i32 = np.arange(0, _DMODEL, 2, dtype=np.float32)
    den = np.repeat(np.power(10000.0, i32 / float(_DMODEL)).astype(np.float64), 2)
    c = np.arange(_DMODEL)
    even = (c % 2 == 0)[None, :]
    u = np.arange(_NU, dtype=np.float64)[:, None]
    t = np.arange(_CH, dtype=np.float64)[:, None]
    a_ang = (u * _CH) / den[None, :]
    b_ang = t / den[None, :]
    a1 = np.where(even, np.sin(a_ang), np.cos(a_ang)).astype(np.float32)
    a2 = np.where(even, np.cos(a_ang), -np.sin(a_ang)).astype(np.float32)

    def pack(x):
        g = x.astype(np.float32).astype(ml_dtypes.bfloat16)
        bits = g.reshape(_CH, _NGRP, 2, _L).view(np.uint16).astype(np.uint32)
        return (bits[:, :, 0, :] | (bits[:, :, 1, :] << 16)).reshape(-1)

    a12 = np.concatenate([a1, a2], axis=1).reshape(-1)
    b12 = np.concatenate([pack(np.cos(b_ang)), pack(np.sin(b_ang))])
    return a12, b12.view(np.int32)


_A12, _B12 = _pe_factors()

_mesh = plsc.VectorSubcoreMesh(core_axis_name="c", subcore_axis_name="s")


@pl.kernel(
    mesh=_mesh,
    out_type=jax.ShapeDtypeStruct((_BATCH, _SEQ, _DMODEL), jnp.float32),
    scratch_types=[
        pltpu.VMEM((_BATCH * _SPW,), jnp.int32),
        pltpu.VMEM((2 * _DMODEL,), jnp.float32),
        pltpu.VMEM((2 * _BWORDS,), jnp.int32),
        pltpu.VMEM((_CH, _DMODEL), jnp.float32),
        pltpu.VMEM((_CH, _DMODEL), jnp.float32),
        pltpu.SemaphoreType.DMA,
        pltpu.SemaphoreType.DMA,
        pltpu.SemaphoreType.DMA,
        pltpu.SemaphoreType.DMA,
    ],
)
def _bi_embed(x_hbm, table_hbm, a_hbm, b_hbm, out_hbm,
              idx_v, a_v, b_v, rows_a, rows_b, sg_a, sg_b, sw_a, sw_b):
    wid = lax.axis_index("s") * _NC + lax.axis_index("c")
    s0 = wid * _SPW
    u0 = wid * 2

    for b in range(_BATCH):
        pltpu.sync_copy(x_hbm.at[b, pl.ds(s0, _SPW)],
                        idx_v.at[pl.ds(b * _SPW, _SPW)])

    bufs = (rows_a, rows_b)
    gsems = (sg_a, sg_b)
    wsems = (sw_a, sw_b)

    def start_gather(k):
        idx_off = (k % _BATCH) * _SPW + (k // _BATCH) * _CH
        return pltpu.async_copy(
            table_hbm.at[idx_v.at[pl.ds(idx_off, _CH)]],
            bufs[k % 2], gsems[k % 2])

    pltpu.sync_copy(b_hbm, b_v)
    pltpu.sync_copy(a_hbm.at[pl.ds(u0 * (2 * _DMODEL), 2 * _DMODEL)], a_v)
    g = start_gather(0)
    pending_w = [None, None]

    for k in range(_NSTEP):
        nxt = None
        if k + 1 < _NSTEP:
            if pending_w[(k + 1) % 2] is not None:
                pending_w[(k + 1) % 2].wait()
                pending_w[(k + 1) % 2] = None
            nxt = start_gather(k + 1)
        g.wait()
        buf = bufs[k % 2]

        def group(gi, carry):
            base = gi * 2 * _L
            a1lo = a_v[pl.ds(base, _L)]
            a1hi = a_v[pl.ds(base + _L, _L)]
            a2lo = a_v[pl.ds(_DMODEL + base, _L)]
            a2hi = a_v[pl.ds(_DMODEL + base + _L, _L)]

            @plsc.parallel_loop(0, _CH, unroll=4)
            def _row(t):
                w1 = b_v[pl.ds(t * (_DMODEL // 2) + gi * _L, _L)]
                w2 = b_v[pl.ds(_BWORDS + t * (_DMODEL // 2) + gi * _L, _L)]
                b1lo = lax.bitcast_convert_type(w1 << 16, jnp.float32)
                b1hi = lax.bitcast_convert_type(w1 & jnp.int32(-65536),
                                                jnp.float32)
                b2lo = lax.bitcast_convert_type(w2 << 16, jnp.float32)
                b2hi = lax.bitcast_convert_type(w2 & jnp.int32(-65536),
                                                jnp.float32)
                plsc.addupdate(buf.at[t, pl.ds(base, _L)],
                               a1lo * b1lo + a2lo * b2lo)
                plsc.addupdate(buf.at[t, pl.ds(base + _L, _L)],
                               a1hi * b1hi + a2hi * b2hi)

            return carry

        lax.fori_loop(0, _NGRP, group, 0)

        out_s = s0 + (k // _BATCH) * _CH
        pending_w[k % 2] = pltpu.async_copy(
            buf, out_hbm.at[k % _BATCH, pl.ds(out_s, _CH)], wsems[k % 2])

        if k == _BATCH - 1:
            pltpu.sync_copy(
                a_hbm.at[pl.ds((u0 + 1) * (2 * _DMODEL), 2 * _DMODEL)], a_v)
        g = nxt

    for w in pending_w:
        if w is not None:
            w.wait()


def kernel(x, table):
    return _bi_embed(x, table, jnp.asarray(_A12), jnp.asarray(_B12))


# speedup vs baseline: 1.0071x; 1.0071x over previous
import numpy as np
import jax
import jax.numpy as jnp
from jax import lax
from jax.experimental import pallas as pl
from jax.experimental.pallas import tpu as pltpu
from jax.experimental.pallas import tpu_sc as plsc

_VOCAB, _DMODEL, _BATCH, _SEQ = 100000, 1024, 4, 2048
_NC, _NS, _L = 2, 16, 16
_NW = _NC * _NS
_B = _BATCH * _SEQ
_SPW = _SEQ // _NW
_CH = 32
_NSTEP = (_SPW // _CH) * _BATCH
_NGRP = _DMODEL // (2 * _L)
_NU = _SEQ // _CH
_WPR = _DMODEL // 2
_BWORDS = _CH * _WPR
_AWORDS = _NU * 2 * _DMODEL
_NPAIR = _CH * _WPR // _L
_MASKHI = jnp.int32(-65536)


def _pe_factors():
    import ml_dtypes
    i32 = np.arange(0, _DMODEL, 2, dtype=np.float32)
    den = np.repeat(np.power(10000.0, i32 / float(_DMODEL)).astype(np.float64), 2)
    c = np.arange(_DMODEL)
    even = (c % 2 == 0)[None, :]
    u = np.arange(_NU, dtype=np.float64)[:, None]
    t = np.arange(_CH, dtype=np.float64)[:, None]
    a_ang = (u * _CH) / den[None, :]
    b_ang = t / den[None, :]
    a1 = np.where(even, np.sin(a_ang), np.cos(a_ang)).astype(np.float32)
    a2 = np.where(even, np.cos(a_ang), -np.sin(a_ang)).astype(np.float32)

    def pack(x):
        g = x.astype(np.float32).astype(ml_dtypes.bfloat16)
        bits = g.reshape(_CH, _NGRP, 2, _L).view(np.uint16).astype(np.uint32)
        return (bits[:, :, 0, :] | (bits[:, :, 1, :] << 16)).reshape(-1)

    a12 = np.concatenate([a1, a2], axis=1).reshape(-1).view(np.uint32)
    b12 = np.concatenate([pack(np.cos(b_ang)), pack(np.sin(b_ang))])
    return np.concatenate([a12, b12]).view(np.int32)


_AB = _pe_factors()

_mesh = plsc.VectorSubcoreMesh(core_axis_name="c", subcore_axis_name="s")


@pl.kernel(
    mesh=_mesh,
    out_type=jax.ShapeDtypeStruct((_BATCH, _SEQ, _DMODEL), jnp.float32),
    scratch_types=[
        pltpu.VMEM((_BATCH * _SPW,), jnp.int32),
        pltpu.VMEM((2 * _DMODEL,), jnp.int32),
        pltpu.VMEM((2 * _BWORDS,), jnp.int32),
        pltpu.VMEM((_CH * _WPR,), jnp.int32),
        pltpu.VMEM((_CH, _DMODEL), jnp.float32),
        pltpu.VMEM((_CH, _DMODEL), jnp.float32),
        pltpu.SemaphoreType.DMA,
        pltpu.SemaphoreType.DMA,
        pltpu.SemaphoreType.DMA,
        pltpu.SemaphoreType.DMA,
    ],
)
def _bi_embed(x_hbm, table_hbm, ab_hbm, out_hbm,
              idx_v, a_v, b_v, pe_v, rows_a, rows_b, sg_a, sg_b, sw_a, sw_b):
    wid = lax.axis_index("s") * _NC + lax.axis_index("c")
    s0 = wid * _SPW
    u0 = wid * 2

    for b in range(_BATCH):
        pltpu.sync_copy(x_hbm.at[b, pl.ds(s0, _SPW)],
                        idx_v.at[pl.ds(b * _SPW, _SPW)])

    bufs = (rows_a, rows_b)
    gsems = (sg_a, sg_b)
    wsems = (sw_a, sw_b)

    def start_gather(k):
        idx_off = (k % _BATCH) * _SPW + (k // _BATCH) * _CH
        return pltpu.async_copy(
            table_hbm.at[idx_v.at[pl.ds(idx_off, _CH)]],
            bufs[k % 2], gsems[k % 2])

    def load_a(u):
        pltpu.sync_copy(ab_hbm.at[pl.ds(u * (2 * _DMODEL), 2 * _DMODEL)], a_v)

    def build_pe():
        def group(gi, carry):
            base = gi * 2 * _L
            a1lo = lax.bitcast_convert_type(a_v[pl.ds(base, _L)], jnp.float32)
            a1hi = lax.bitcast_convert_type(a_v[pl.ds(base + _L, _L)],
                                            jnp.float32)
            a2lo = lax.bitcast_convert_type(a_v[pl.ds(_DMODEL + base, _L)],
                                            jnp.float32)
            a2hi = lax.bitcast_convert_type(
                a_v[pl.ds(_DMODEL + base + _L, _L)], jnp.float32)

            @plsc.parallel_loop(0, _CH, unroll=4)
            def _row(t):
                off = t * _WPR + gi * _L
                w1 = b_v[pl.ds(off, _L)]
                w2 = b_v[pl.ds(_BWORDS + off, _L)]
                b1lo = lax.bitcast_convert_type(w1 << 16, jnp.float32)
                b1hi = lax.bitcast_convert_type(w1 & _MASKHI, jnp.float32)
                b2lo = lax.bitcast_convert_type(w2 << 16, jnp.float32)
                b2hi = lax.bitcast_convert_type(w2 & _MASKHI, jnp.float32)
                pe_lo = a1lo * b1lo + a2lo * b2lo
                pe_hi = a1hi * b1hi + a2hi * b2hi
                packed = (
                    (lax.bitcast_convert_type(pe_hi, jnp.int32) & _MASKHI)
                    | lax.shift_right_logical(
                        lax.bitcast_convert_type(pe_lo, jnp.int32), 16))
                pe_v[pl.ds(off, _L)] = packed

            return carry

        lax.fori_loop(0, _NGRP, group, 0)

    g = start_gather(0)
    pltpu.sync_copy(ab_hbm.at[pl.ds(_AWORDS, 2 * _BWORDS)], b_v)
    load_a(u0)
    build_pe()
    pending_w = [None, None]

    for k in range(_NSTEP):
        nxt = None
        if k + 1 < _NSTEP:
            if pending_w[(k + 1) % 2] is not None:
                pending_w[(k + 1) % 2].wait()
                pending_w[(k + 1) % 2] = None
            nxt = start_gather(k + 1)
        g.wait()
        buf = bufs[k % 2]

        @plsc.parallel_loop(0, _NPAIR, unroll=4)
        def _add(i):
            r = i // _NGRP
            c = (i % _NGRP) * 2 * _L
            w = pe_v[pl.ds(i * _L, _L)]
            lo = lax.bitcast_convert_type(w << 16, jnp.float32)
            hi = lax.bitcast_convert_type(w & _MASKHI, jnp.float32)
            plsc.addupdate(buf.at[r, pl.ds(c, _L)], lo)
            plsc.addupdate(buf.at[r, pl.ds(c + _L, _L)], hi)

        out_s = s0 + (k // _BATCH) * _CH
        pending_w[k % 2] = pltpu.async_copy(
            buf, out_hbm.at[k % _BATCH, pl.ds(out_s, _CH)], wsems[k % 2])

        if k == _BATCH - 1:
            load_a(u0 + 1)
            build_pe()
        g = nxt

    for w in pending_w:
        if w is not None:
            w.wait()


def kernel(x, table):
    return _bi_embed(x, table, jnp.asarray(_AB))


# speedup vs baseline: 1.1473x; 1.1392x over previous
import numpy as np
import jax
import jax.numpy as jnp
from jax import lax
from jax.experimental import pallas as pl
from jax.experimental.pallas import tpu as pltpu
from jax.experimental.pallas import tpu_sc as plsc

_VOCAB, _DMODEL, _BATCH, _SEQ = 100000, 1024, 4, 2048
_NC, _NS, _L = 2, 16, 16
_NW = _NC * _NS
_B = _BATCH * _SEQ
_SPW = _SEQ // _NW
_CH = 16
_NSC = _SPW // _CH
_NSTEP = _NSC * _BATCH
_NBUF = 3
_WPR = _DMODEL // 2
_NPAIR = _CH * _WPR // _L
_GPR = _WPR // _L
_MASKHI = jnp.int32(-65536)


def _pe_table(seq_len, d):
    pos = np.arange(seq_len, dtype=np.float32)[:, None]
    i = np.arange(0, d, 2, dtype=np.float32)[None, :]
    angle = pos / np.power(10000.0, i / float(d))
    pe = np.zeros((seq_len, d), dtype=np.float32)
    pe[:, 0::2] = np.sin(angle)
    pe[:, 1::2] = np.cos(angle)
    return pe


def _pe_bf16_packed():
    import ml_dtypes
    pe = _pe_table(_SEQ, _DMODEL).reshape(-1, 2, 16)
    bits = pe.astype(ml_dtypes.bfloat16).view(np.uint16).astype(np.uint32)
    words = bits[:, 0, :] | (bits[:, 1, :] << 16)
    return words.reshape(-1).view(np.int32)


_PE = _pe_bf16_packed()

_mesh = plsc.VectorSubcoreMesh(core_axis_name="c", subcore_axis_name="s")


@pl.kernel(
    mesh=_mesh,
    out_type=jax.ShapeDtypeStruct((_BATCH, _SEQ, _DMODEL), jnp.float32),
    scratch_types=(
        [pltpu.VMEM((_BATCH * _SPW,), jnp.int32),
         pltpu.VMEM((_CH * _WPR,), jnp.int32)]
        + [pltpu.VMEM((_CH, _DMODEL), jnp.float32)] * _NBUF
        + [pltpu.SemaphoreType.DMA] * (2 * _NBUF)
    ),
)
def _bi_embed(x_hbm, table_hbm, pe_hbm, out_hbm, idx_v, pe_v, *bufs_sems):
    bufs = bufs_sems[:_NBUF]
    gsems = bufs_sems[_NBUF:2 * _NBUF]
    wsems = bufs_sems[2 * _NBUF:]
    wid = lax.axis_index("s") * _NC + lax.axis_index("c")
    s0 = wid * _SPW

    for b in range(_BATCH):
        pltpu.sync_copy(x_hbm.at[b, pl.ds(s0, _SPW)],
                        idx_v.at[pl.ds(b * _SPW, _SPW)])

    def start_gather(k):
        idx_off = (k % _BATCH) * _SPW + (k // _BATCH) * _CH
        return pltpu.async_copy(
            table_hbm.at[idx_v.at[pl.ds(idx_off, _CH)]],
            bufs[k % _NBUF], gsems[k % _NBUF])

    def load_pe(sc):
        pltpu.sync_copy(
            pe_hbm.at[pl.ds((s0 + sc * _CH) * _WPR, _CH * _WPR)], pe_v)

    g = [None] * _NBUF
    g[0] = start_gather(0)
    load_pe(0)
    g[1] = start_gather(1)
    pending_w = [None] * _NBUF

    for k in range(_NSTEP):
        if k + 2 < _NSTEP:
            j = (k + 2) % _NBUF
            if pending_w[j] is not None:
                pending_w[j].wait()
                pending_w[j] = None
            g[j] = start_gather(k + 2)
        g[k % _NBUF].wait()
        buf = bufs[k % _NBUF]

        @plsc.parallel_loop(0, _NPAIR, unroll=4)
        def _add(i):
            r = i // _GPR
            c = (i % _GPR) * 2 * _L
            w = pe_v[pl.ds(i * _L, _L)]
            lo = lax.bitcast_convert_type(w << 16, jnp.float32)
            hi = lax.bitcast_convert_type(w & _MASKHI, jnp.float32)
            plsc.addupdate(buf.at[r, pl.ds(c, _L)], lo)
            plsc.addupdate(buf.at[r, pl.ds(c + _L, _L)], hi)

        out_s = s0 + (k // _BATCH) * _CH
        pending_w[k % _NBUF] = pltpu.async_copy(
            buf, out_hbm.at[k % _BATCH, pl.ds(out_s, _CH)], wsems[k % _NBUF])

        if (k + 1) % _BATCH == 0 and k + 1 < _NSTEP:
            load_pe((k + 1) // _BATCH)

    for w in pending_w:
        if w is not None:
            w.wait()


def kernel(x, table):
    pe = jnp.asarray(_PE)
    return _bi_embed(x, table, pe)


# speedup vs baseline: 1.2228x; 1.0658x over previous
import numpy as np
import jax
import jax.numpy as jnp
from jax import lax
from jax.experimental import pallas as pl
from jax.experimental.pallas import tpu as pltpu
from jax.experimental.pallas import tpu_sc as plsc

_VOCAB, _DMODEL, _BATCH, _SEQ = 100000, 1024, 4, 2048
_NC, _NS, _L = 2, 16, 16
_NW = _NC * _NS
_B = _BATCH * _SEQ
_SPW = _SEQ // _NW
_CH = 32
_NSC = _SPW // _CH
_NSTEP = _NSC * _BATCH
_NBUF = 3
_WPR = _DMODEL // 2
_NPAIR = _CH * _WPR // _L
_GPR = _WPR // _L
_MASKHI = jnp.int32(-65536)


def _pe_table(seq_len, d):
    pos = np.arange(seq_len, dtype=np.float32)[:, None]
    i = np.arange(0, d, 2, dtype=np.float32)[None, :]
    angle = pos / np.power(10000.0, i / float(d))
    pe = np.zeros((seq_len, d), dtype=np.float32)
    pe[:, 0::2] = np.sin(angle)
    pe[:, 1::2] = np.cos(angle)
    return pe


def _pe_bf16_packed():
    import ml_dtypes
    pe = _pe_table(_SEQ, _DMODEL).reshape(-1, 2, 16)
    bits = pe.astype(ml_dtypes.bfloat16).view(np.uint16).astype(np.uint32)
    words = bits[:, 0, :] | (bits[:, 1, :] << 16)
    return words.reshape(-1).view(np.int32)


_PE = _pe_bf16_packed()

_mesh = plsc.VectorSubcoreMesh(core_axis_name="c", subcore_axis_name="s")


@pl.kernel(
    mesh=_mesh,
    out_type=jax.ShapeDtypeStruct((_BATCH, _SEQ, _DMODEL), jnp.float32),
    scratch_types=(
        [pltpu.VMEM((_BATCH * _SPW,), jnp.int32),
         pltpu.VMEM((_CH * _WPR,), jnp.int32)]
        + [pltpu.VMEM((_CH, _DMODEL), jnp.float32)] * _NBUF
        + [pltpu.SemaphoreType.DMA] * (2 * _NBUF)
    ),
)
def _bi_embed(x_hbm, table_hbm, pe_hbm, out_hbm, idx_v, pe_v, *bufs_sems):
    bufs = bufs_sems[:_NBUF]
    gsems = bufs_sems[_NBUF:2 * _NBUF]
    wsems = bufs_sems[2 * _NBUF:]
    wid = lax.axis_index("s") * _NC + lax.axis_index("c")
    s0 = wid * _SPW

    for b in range(_BATCH):
        pltpu.sync_copy(x_hbm.at[b, pl.ds(s0, _SPW)],
                        idx_v.at[pl.ds(b * _SPW, _SPW)])

    def start_gather(k):
        idx_off = (k % _BATCH) * _SPW + (k // _BATCH) * _CH
        return pltpu.async_copy(
            table_hbm.at[idx_v.at[pl.ds(idx_off, _CH)]],
            bufs[k % _NBUF], gsems[k % _NBUF])

    def load_pe(sc):
        pltpu.sync_copy(
            pe_hbm.at[pl.ds((s0 + sc * _CH) * _WPR, _CH * _WPR)], pe_v)

    g = [None] * _NBUF
    g[0] = start_gather(0)
    load_pe(0)
    g[1] = start_gather(1)
    pending_w = [None] * _NBUF

    for k in range(_NSTEP):
        if k + 2 < _NSTEP:
            j = (k + 2) % _NBUF
            if pending_w[j] is not None:
                pending_w[j].wait()
                pending_w[j] = None
            g[j] = start_gather(k + 2)
        g[k % _NBUF].wait()
        buf = bufs[k % _NBUF]

        @plsc.parallel_loop(0, _NPAIR, unroll=4)
        def _add(i):
            r = i // _GPR
            c = (i % _GPR) * 2 * _L
            w = pe_v[pl.ds(i * _L, _L)]
            lo = lax.bitcast_convert_type(w << 16, jnp.float32)
            hi = lax.bitcast_convert_type(w & _MASKHI, jnp.float32)
            plsc.addupdate(buf.at[r, pl.ds(c, _L)], lo)
            plsc.addupdate(buf.at[r, pl.ds(c + _L, _L)], hi)

        out_s = s0 + (k // _BATCH) * _CH
        pending_w[k % _NBUF] = pltpu.async_copy(
            buf, out_hbm.at[k % _BATCH, pl.ds(out_s, _CH)], wsems[k % _NBUF])

        if (k + 1) % _BATCH == 0 and k + 1 < _NSTEP:
            load_pe((k + 1) // _BATCH)

    for w in pending_w:
        if w is not None:
            w.wait()


def kernel(x, table):
    pe = jnp.asarray(_PE)
    return _bi_embed(x, table, pe)
